# Initial kernel scaffold; baseline (speedup 1.0000x reference)
import jax
import jax.numpy as jnp
from jax.experimental import pallas as pl


def kernel(binary_list, eval, weight_initial, weights):
    raise NotImplementedError("write your pallas kernel here")


# speedup vs baseline: 365.2699x; 365.2699x over previous
import functools

import jax
import jax.numpy as jnp
from jax import lax
from jax.experimental import pallas as pl
from jax.experimental.pallas import tpu as pltpu
from jax.experimental.pallas import tpu_sc as plsc

B, L = 16384, 64
NC, NS = 2, 16
NW = NC * NS
ROWS = B // NW
LANES = 16
GPL = 4
CHUNK = LANES * GPL
NCHUNK = ROWS // CHUNK

_mesh = plsc.VectorSubcoreMesh(
    core_axis_name="c", subcore_axis_name="s", num_cores=NC, num_subcores=NS
)


@functools.partial(
    pl.kernel,
    out_type=(
        jax.ShapeDtypeStruct((B,), jnp.float32),
        jax.ShapeDtypeStruct((B,), jnp.int32),
    ),
    mesh=_mesh,
    scratch_types=(
        pltpu.VMEM((ROWS * L,), jnp.int32),
        pltpu.VMEM((5, LANES), jnp.float32),
        pltpu.VMEM((2, LANES), jnp.int32),
        pltpu.VMEM((ROWS,), jnp.float32),
        pltpu.VMEM((ROWS,), jnp.int32),
    ),
    compiler_params=pltpu.CompilerParams(needs_layout_passes=False),
)
def _sc_walk(bits_hbm, pf_hbm, pi_hbm, pred_hbm, sym_hbm,
             bits_v, pf_v, pi_v, pred_v, sym_v):
    wid = lax.axis_index("s") * NC + lax.axis_index("c")
    base = wid * ROWS
    pltpu.sync_copy(pf_hbm, pf_v)
    pltpu.sync_copy(pi_hbm, pi_v)
    pltpu.sync_copy(bits_hbm.at[pl.ds(base * L, ROWS * L)], bits_v)

    mpack = pi_v[0]
    s0 = pi_v[1]
    ftv = pf_v[4]
    tv = [pf_v[j] for j in range(4)]
    big = jnp.full((LANES,), 3.4e38, jnp.float32)
    one = jnp.full((LANES,), 1, jnp.int32)
    zero = jnp.zeros((LANES,), jnp.int32)
    lane = lax.iota(jnp.int32, LANES)

    for c in range(NCHUNK):
        rowoff = [(lane + (c * CHUNK + g * LANES)) * L for g in range(GPL)]

        def step(i, carry, rowoff=rowoff):
            ss, vv = carry
            col = jnp.full((LANES,), i, jnp.int32)
            nss, nvv = [], []
            for g in range(GPL):
                a2 = plsc.load_gather(bits_v, [rowoff[g] + col])
                a2 = a2 + a2
                s = (mpack >> (a2 + ss[g])) & one
                nss.append(s)
                nvv.append(vv[g] | (one << (a2 + s)))
            return tuple(nss), tuple(nvv)

        init = (tuple(s0 for _ in range(GPL)), tuple(zero for _ in range(GPL)))
        ss, vv = lax.fori_loop(0, L, step, init)

        for g in range(GPL):
            pred = ftv
            for j in range(4):
                hit = ((vv[g] >> j) & one) == one
                pred = jnp.minimum(pred, jnp.where(hit, tv[j], big))
            off = c * CHUNK + g * LANES
            pred_v[pl.ds(off, LANES)] = pred
            sym_v[pl.ds(off, LANES)] = ss[g]

    pltpu.sync_copy(pred_v, pred_hbm.at[pl.ds(base, ROWS)])
    pltpu.sync_copy(sym_v, sym_hbm.at[pl.ds(base, ROWS)])


def kernel(binary_list, eval, weight_initial, weights):
    sw = jax.nn.softmax(weights, axis=-1)
    truths = jnp.max(sw, axis=-1).reshape(-1)
    matrix = jnp.argmax(sw, axis=-1).astype(jnp.int32).reshape(-1)
    p0 = jax.nn.softmax(weight_initial, axis=1)
    ftv = jnp.max(p0)
    s0 = jnp.argmax(p0).astype(jnp.int32)
    mpack = matrix[0] | (matrix[1] << 1) | (matrix[2] << 2) | (matrix[3] << 3)
    pf = jnp.broadcast_to(
        jnp.concatenate([truths, ftv[None]])[:, None], (5, LANES)
    ).astype(jnp.float32)
    pi = jnp.broadcast_to(
        jnp.stack([mpack, s0])[:, None], (2, LANES)
    ).astype(jnp.int32)
    pred, sym = _sc_walk(binary_list.reshape(-1), pf, pi)
    return pred, sym.reshape(B, 1, 1)


# speedup vs baseline: 390.6461x; 1.0695x over previous
import functools

import jax
import jax.numpy as jnp
from jax import lax
from jax.experimental import pallas as pl
from jax.experimental.pallas import tpu as pltpu
from jax.experimental.pallas import tpu_sc as plsc

B, L = 16384, 64
NC, NS = 2, 16
NW = NC * NS
ROWS = B // NW
LANES = 16
GPL = 8
CHUNK = LANES * GPL
NCHUNK = ROWS // CHUNK

_mesh = plsc.VectorSubcoreMesh(
    core_axis_name="c", subcore_axis_name="s", num_cores=NC, num_subcores=NS
)


@functools.partial(
    pl.kernel,
    out_type=(
        jax.ShapeDtypeStruct((B,), jnp.float32),
        jax.ShapeDtypeStruct((B,), jnp.int32),
    ),
    mesh=_mesh,
    scratch_types=(
        pltpu.VMEM((ROWS, L), jnp.int32),
        pltpu.VMEM((5, LANES), jnp.float32),
        pltpu.VMEM((2, LANES), jnp.int32),
        pltpu.VMEM((ROWS,), jnp.float32),
        pltpu.VMEM((ROWS,), jnp.int32),
    ),
    compiler_params=pltpu.CompilerParams(needs_layout_passes=False),
)
def _sc_walk(bits_hbm, pf_hbm, pi_hbm, pred_hbm, sym_hbm,
             bits_v, pf_v, pi_v, pred_v, sym_v):
    wid = lax.axis_index("s") * NC + lax.axis_index("c")
    base = wid * ROWS
    pltpu.sync_copy(pf_hbm, pf_v)
    pltpu.sync_copy(pi_hbm, pi_v)
    pltpu.sync_copy(bits_hbm.at[pl.ds(base, ROWS)], bits_v)

    mpack = pi_v[0]
    s0 = pi_v[1]
    ftv = pf_v[4]
    tv = [pf_v[j] for j in range(4)]
    big = jnp.full((LANES,), 3.4e38, jnp.float32)
    one = jnp.full((LANES,), 1, jnp.int32)
    zero = jnp.zeros((LANES,), jnp.int32)
    lane = lax.iota(jnp.int32, LANES)

    for c in range(NCHUNK):
        rows = [lane + (c * CHUNK + g * LANES) for g in range(GPL)]

        def step(i, carry, rows=rows):
            ss, vv = carry
            col = jnp.full((LANES,), i, jnp.int32)
            nss, nvv = [], []
            for g in range(GPL):
                a2 = plsc.load_gather(bits_v, [rows[g], col])
                a2 = a2 + a2
                s = (mpack >> (a2 + ss[g])) & one
                nss.append(s)
                nvv.append(vv[g] | (one << (a2 + s)))
            return tuple(nss), tuple(nvv)

        init = (tuple(s0 for _ in range(GPL)), tuple(zero for _ in range(GPL)))
        ss, vv = lax.fori_loop(0, L, step, init)

        for g in range(GPL):
            pred = ftv
            for j in range(4):
                hit = ((vv[g] >> j) & one) == one
                pred = jnp.minimum(pred, jnp.where(hit, tv[j], big))
            off = c * CHUNK + g * LANES
            pred_v[pl.ds(off, LANES)] = pred
            sym_v[pl.ds(off, LANES)] = ss[g]

    pltpu.sync_copy(pred_v, pred_hbm.at[pl.ds(base, ROWS)])
    pltpu.sync_copy(sym_v, sym_hbm.at[pl.ds(base, ROWS)])


def kernel(binary_list, eval, weight_initial, weights):
    sw = jax.nn.softmax(weights, axis=-1)
    truths = jnp.max(sw, axis=-1).reshape(-1)
    matrix = jnp.argmax(sw, axis=-1).astype(jnp.int32).reshape(-1)
    p0 = jax.nn.softmax(weight_initial, axis=1)
    ftv = jnp.max(p0)
    s0 = jnp.argmax(p0).astype(jnp.int32)
    mpack = matrix[0] | (matrix[1] << 1) | (matrix[2] << 2) | (matrix[3] << 3)
    pf = jnp.broadcast_to(
        jnp.concatenate([truths, ftv[None]])[:, None], (5, LANES)
    ).astype(jnp.float32)
    pi = jnp.broadcast_to(
        jnp.stack([mpack, s0])[:, None], (2, LANES)
    ).astype(jnp.int32)
    pred, sym = _sc_walk(binary_list, pf, pi)
    return pred, sym.reshape(B, 1, 1)


# speedup vs baseline: 637.6122x; 1.6322x over previous
import functools

import jax
import jax.numpy as jnp
from jax import lax
from jax.experimental import pallas as pl
from jax.experimental.pallas import tpu as pltpu
from jax.experimental.pallas import tpu_sc as plsc

B, L = 16384, 64
NC, NS = 2, 16
NW = NC * NS
ROWS = B // NW
LANES = 16
GPL = 8
CHUNK = LANES * GPL
NCHUNK = ROWS // CHUNK

_mesh = plsc.VectorSubcoreMesh(
    core_axis_name="c", subcore_axis_name="s", num_cores=NC, num_subcores=NS
)


@functools.partial(
    pl.kernel,
    out_type=(
        jax.ShapeDtypeStruct((B,), jnp.float32),
        jax.ShapeDtypeStruct((B,), jnp.int32),
    ),
    mesh=_mesh,
    scratch_types=(
        pltpu.VMEM((L, ROWS), jnp.int32),
        pltpu.VMEM((5, LANES), jnp.float32),
        pltpu.VMEM((2, LANES), jnp.int32),
        pltpu.VMEM((ROWS,), jnp.float32),
        pltpu.VMEM((ROWS,), jnp.int32),
    ),
    compiler_params=pltpu.CompilerParams(needs_layout_passes=False),
)
def _sc_walk(bits_hbm, pf_hbm, pi_hbm, pred_hbm, sym_hbm,
             bits_v, pf_v, pi_v, pred_v, sym_v):
    wid = lax.axis_index("s") * NC + lax.axis_index("c")
    base = wid * ROWS
    pltpu.sync_copy(pf_hbm, pf_v)
    pltpu.sync_copy(pi_hbm, pi_v)
    pltpu.sync_copy(bits_hbm.at[:, pl.ds(base, ROWS)], bits_v)

    mpack = pi_v[0]
    s0 = pi_v[1]
    ftv = pf_v[4]
    tv = [pf_v[j] for j in range(4)]
    big = jnp.full((LANES,), 3.4e38, jnp.float32)
    one = jnp.full((LANES,), 1, jnp.int32)
    zero = jnp.zeros((LANES,), jnp.int32)
    lane = lax.iota(jnp.int32, LANES)

    for c in range(NCHUNK):
        rows = [lane + (c * CHUNK + g * LANES) for g in range(GPL)]

        def step(i, carry, rows=rows):
            ss, vv = carry
            col = jnp.full((LANES,), i, jnp.int32)
            nss, nvv = [], []
            for g in range(GPL):
                a2 = plsc.load_gather(bits_v, [col, rows[g]])
                a2 = a2 + a2
                s = (mpack >> (a2 + ss[g])) & one
                nss.append(s)
                nvv.append(vv[g] | (one << (a2 + s)))
            return tuple(nss), tuple(nvv)

        init = (tuple(s0 for _ in range(GPL)), tuple(zero for _ in range(GPL)))
        ss, vv = lax.fori_loop(0, L, step, init)

        for g in range(GPL):
            pred = ftv
            for j in range(4):
                hit = ((vv[g] >> j) & one) == one
                pred = jnp.minimum(pred, jnp.where(hit, tv[j], big))
            off = c * CHUNK + g * LANES
            pred_v[pl.ds(off, LANES)] = pred
            sym_v[pl.ds(off, LANES)] = ss[g]

    pltpu.sync_copy(pred_v, pred_hbm.at[pl.ds(base, ROWS)])
    pltpu.sync_copy(sym_v, sym_hbm.at[pl.ds(base, ROWS)])


def kernel(binary_list, eval, weight_initial, weights):
    sw = jax.nn.softmax(weights, axis=-1)
    truths = jnp.max(sw, axis=-1).reshape(-1)
    matrix = jnp.argmax(sw, axis=-1).astype(jnp.int32).reshape(-1)
    p0 = jax.nn.softmax(weight_initial, axis=1)
    ftv = jnp.max(p0)
    s0 = jnp.argmax(p0).astype(jnp.int32)
    mpack = matrix[0] | (matrix[1] << 1) | (matrix[2] << 2) | (matrix[3] << 3)
    pf = jnp.broadcast_to(
        jnp.concatenate([truths, ftv[None]])[:, None], (5, LANES)
    ).astype(jnp.float32)
    pi = jnp.broadcast_to(
        jnp.stack([mpack, s0])[:, None], (2, LANES)
    ).astype(jnp.int32)
    pred, sym = _sc_walk(binary_list.T, pf, pi)
    return pred, sym.reshape(B, 1, 1)


# speedup vs baseline: 897.1199x; 1.4070x over previous
import functools

import jax
import jax.numpy as jnp
from jax import lax
from jax.experimental import pallas as pl
from jax.experimental.pallas import tpu as pltpu
from jax.experimental.pallas import tpu_sc as plsc

B, L = 16384, 64
NC, NS = 2, 16
NW = NC * NS
ROWS = B // NW
LANES = 16
GPL = 8
CHUNK = LANES * GPL
NCHUNK = ROWS // CHUNK

_mesh = plsc.VectorSubcoreMesh(
    core_axis_name="c", subcore_axis_name="s", num_cores=NC, num_subcores=NS
)


@functools.partial(
    pl.kernel,
    out_type=(
        jax.ShapeDtypeStruct((B,), jnp.float32),
        jax.ShapeDtypeStruct((B,), jnp.int32),
    ),
    mesh=_mesh,
    scratch_types=(
        pltpu.VMEM((L, ROWS), jnp.int32),
        pltpu.VMEM((LANES,), jnp.float32),
        pltpu.VMEM((LANES,), jnp.float32),
        pltpu.VMEM((LANES,), jnp.int32),
        pltpu.VMEM((ROWS,), jnp.float32),
        pltpu.VMEM((ROWS,), jnp.int32),
    ),
    compiler_params=pltpu.CompilerParams(needs_layout_passes=False),
)
def _sc_walk(bits_hbm, pw_hbm, pred_hbm, sym_hbm,
             bits_v, pw_v, t_v, m_v, pred_v, sym_v):
    wid = lax.axis_index("s") * NC + lax.axis_index("c")
    base = wid * ROWS
    pltpu.sync_copy(pw_hbm, pw_v)
    pltpu.sync_copy(bits_hbm.at[:, pl.ds(base, ROWS)], bits_v)

    big = jnp.full((LANES,), 3.4e38, jnp.float32)
    one = jnp.full((LANES,), 1, jnp.int32)
    zero = jnp.zeros((LANES,), jnp.int32)
    fone = jnp.full((LANES,), 1.0, jnp.float32)
    lane = lax.iota(jnp.int32, LANES)

    x0 = plsc.load_gather(pw_v, [(lane + lane) & 15])
    x1 = plsc.load_gather(pw_v, [(lane + lane + one) & 15])
    d = x0 - x1
    t = fone / (fone + jnp.exp(-jnp.abs(d)))
    m = jnp.where(x1 > x0, one, zero)
    t_v[...] = t
    m_v[...] = m

    def bcast_f(j):
        return plsc.load_gather(t_v, [jnp.full((LANES,), j, jnp.int32)])

    def bcast_i(j):
        return plsc.load_gather(m_v, [jnp.full((LANES,), j, jnp.int32)])

    ftv = bcast_f(0)
    tv = [bcast_f(j + 1) for j in range(4)]
    s0 = bcast_i(0)
    mb = [bcast_i(k + 1) for k in range(4)]
    mpack = mb[0] | (mb[1] << 1) | (mb[2] << 2) | (mb[3] << 3)

    for c in range(NCHUNK):
        rows = [lane + (c * CHUNK + g * LANES) for g in range(GPL)]

        def step(i, carry, rows=rows):
            ss, vv = carry
            col = jnp.full((LANES,), i, jnp.int32)
            nss, nvv = [], []
            for g in range(GPL):
                a2 = plsc.load_gather(bits_v, [col, rows[g]])
                a2 = a2 + a2
                s = (mpack >> (a2 + ss[g])) & one
                nss.append(s)
                nvv.append(vv[g] | (one << (a2 + s)))
            return tuple(nss), tuple(nvv)

        init = (tuple(s0 for _ in range(GPL)), tuple(zero for _ in range(GPL)))
        ss, vv = lax.fori_loop(0, L, step, init)

        for g in range(GPL):
            pred = ftv
            for j in range(4):
                hit = ((vv[g] >> j) & one) == one
                pred = jnp.minimum(pred, jnp.where(hit, tv[j], big))
            off = c * CHUNK + g * LANES
            pred_v[pl.ds(off, LANES)] = pred
            sym_v[pl.ds(off, LANES)] = ss[g]

    pltpu.sync_copy(pred_v, pred_hbm.at[pl.ds(base, ROWS)])
    pltpu.sync_copy(sym_v, sym_hbm.at[pl.ds(base, ROWS)])


def kernel(binary_list, eval, weight_initial, weights):
    pw = jnp.zeros((LANES,), jnp.float32)
    pw = pw.at[0:2].set(weight_initial.reshape(-1))
    pw = pw.at[2:10].set(weights.reshape(-1))
    pred, sym = _sc_walk(binary_list.T, pw)
    return pred, sym.reshape(B, 1, 1)


# speedup vs baseline: 897.6719x; 1.0006x over previous
import functools

import jax
import jax.numpy as jnp
from jax import lax
from jax.experimental import pallas as pl
from jax.experimental.pallas import tpu as pltpu
from jax.experimental.pallas import tpu_sc as plsc

B, L = 16384, 64
NC, NS = 2, 16
NW = NC * NS
ROWS = B // NW
LANES = 16
GPL = 8
CHUNK = LANES * GPL
NCHUNK = ROWS // CHUNK

_mesh = plsc.VectorSubcoreMesh(
    core_axis_name="c", subcore_axis_name="s", num_cores=NC, num_subcores=NS
)


@functools.partial(
    pl.kernel,
    out_type=(
        jax.ShapeDtypeStruct((B,), jnp.float32),
        jax.ShapeDtypeStruct((B,), jnp.int32),
    ),
    mesh=_mesh,
    scratch_types=(
        pltpu.VMEM((L, ROWS), jnp.int32),
        pltpu.VMEM((10, LANES), jnp.float32),
        pltpu.VMEM((ROWS,), jnp.float32),
        pltpu.VMEM((ROWS,), jnp.int32),
    ),
    compiler_params=pltpu.CompilerParams(needs_layout_passes=False),
)
def _sc_walk(bits_hbm, pw_hbm, pred_hbm, sym_hbm,
             bits_v, pw_v, pred_v, sym_v):
    wid = lax.axis_index("s") * NC + lax.axis_index("c")
    base = wid * ROWS
    pltpu.sync_copy(pw_hbm, pw_v)
    pltpu.sync_copy(bits_hbm.at[:, pl.ds(base, ROWS)], bits_v)

    big = jnp.full((LANES,), 3.4e38, jnp.float32)
    one = jnp.full((LANES,), 1, jnp.int32)
    zero = jnp.zeros((LANES,), jnp.int32)
    fone = jnp.full((LANES,), 1.0, jnp.float32)
    lane = lax.iota(jnp.int32, LANES)

    def pair(k):
        y0 = pw_v[2 * k]
        y1 = pw_v[2 * k + 1]
        t = fone / (fone + jnp.exp(-jnp.abs(y0 - y1)))
        m = jnp.where(y1 > y0, one, zero)
        return t, m

    ftv, s0 = pair(0)
    cells = [pair(k + 1) for k in range(4)]
    tv = [c[0] for c in cells]
    mpack = (cells[0][1] | (cells[1][1] << 1)
             | (cells[2][1] << 2) | (cells[3][1] << 3))

    for c in range(NCHUNK):
        rows = [lane + (c * CHUNK + g * LANES) for g in range(GPL)]

        def step(i, carry, rows=rows):
            ss, vv = carry
            col = jnp.full((LANES,), i, jnp.int32)
            nss, nvv = [], []
            for g in range(GPL):
                a2 = plsc.load_gather(bits_v, [col, rows[g]])
                a2 = a2 + a2
                s = (mpack >> (a2 + ss[g])) & one
                nss.append(s)
                nvv.append(vv[g] | (one << (a2 + s)))
            return tuple(nss), tuple(nvv)

        init = (tuple(s0 for _ in range(GPL)), tuple(zero for _ in range(GPL)))
        ss, vv = lax.fori_loop(0, L, step, init)

        for g in range(GPL):
            pred = ftv
            for j in range(4):
                hit = ((vv[g] >> j) & one) == one
                pred = jnp.minimum(pred, jnp.where(hit, tv[j], big))
            off = c * CHUNK + g * LANES
            pred_v[pl.ds(off, LANES)] = pred
            sym_v[pl.ds(off, LANES)] = ss[g]

    pltpu.sync_copy(pred_v, pred_hbm.at[pl.ds(base, ROWS)])
    pltpu.sync_copy(sym_v, sym_hbm.at[pl.ds(base, ROWS)])


def kernel(binary_list, eval, weight_initial, weights):
    pw = jnp.broadcast_to(
        jnp.concatenate([weight_initial.reshape(-1), weights.reshape(-1)])[:, None],
        (10, LANES),
    )
    pred, sym = _sc_walk(binary_list.T, pw)
    return pred, sym.reshape(B, 1, 1)


# speedup vs baseline: 974.9367x; 1.0861x over previous
import functools

import jax
import jax.numpy as jnp
from jax import lax
from jax.experimental import pallas as pl
from jax.experimental.pallas import tpu as pltpu
from jax.experimental.pallas import tpu_sc as plsc

B, L = 16384, 64
NC, NS = 2, 16
NW = NC * NS
ROWS = B // NW
LANES = 16
KBITS = ROWS // LANES

_mesh = plsc.VectorSubcoreMesh(
    core_axis_name="c", subcore_axis_name="s", num_cores=NC, num_subcores=NS
)


@functools.partial(
    pl.kernel,
    out_type=(
        jax.ShapeDtypeStruct((B,), jnp.float32),
        jax.ShapeDtypeStruct((B,), jnp.int32),
    ),
    mesh=_mesh,
    scratch_types=(
        pltpu.VMEM((L, ROWS), jnp.int32),
        pltpu.VMEM((10, LANES), jnp.float32),
        pltpu.VMEM((ROWS,), jnp.float32),
        pltpu.VMEM((ROWS,), jnp.int32),
    ),
    compiler_params=pltpu.CompilerParams(needs_layout_passes=False),
)
def _sc_walk(bits_hbm, pw_hbm, pred_hbm, sym_hbm,
             bits_v, pw_v, pred_v, sym_v):
    wid = lax.axis_index("s") * NC + lax.axis_index("c")
    base = wid * ROWS
    pltpu.sync_copy(pw_hbm, pw_v)
    pltpu.sync_copy(bits_hbm.at[:, pl.ds(base, ROWS)], bits_v)

    big = jnp.full((LANES,), 3.4e38, jnp.float32)
    one = jnp.full((LANES,), 1, jnp.int32)
    zero = jnp.zeros((LANES,), jnp.int32)
    ones = jnp.full((LANES,), -1, jnp.int32)
    fone = jnp.full((LANES,), 1.0, jnp.float32)
    lane = lax.iota(jnp.int32, LANES)

    def pair(k):
        y0 = pw_v[2 * k]
        y1 = pw_v[2 * k + 1]
        t = fone / (fone + jnp.exp(-jnp.abs(y0 - y1)))
        m = jnp.where(y1 > y0, ones, zero)
        return t, m

    ftv, s0m = pair(0)
    cells = [pair(k + 1) for k in range(4)]
    tv = [c[0] for c in cells]
    m00, m01, m10, m11 = (c[1] for c in cells)

    def step(i, carry):
        st, v00, v01, v10, v11 = carry
        col = jnp.full((LANES,), i, jnp.int32)
        a = zero
        for k in range(KBITS):
            bv = plsc.load_gather(bits_v, [col, lane + (16 * k)])
            a = a | (bv << k)
        na = ~a
        ns = ~st
        nst = (na & ((ns & m00) | (st & m01))) | (a & ((ns & m10) | (st & m11)))
        nns = ~nst
        return (nst,
                v00 | (na & nns), v01 | (na & nst),
                v10 | (a & nns), v11 | (a & nst))

    init = (s0m, zero, zero, zero, zero)
    st, v00, v01, v10, v11 = lax.fori_loop(0, L, step, init)

    vs = (v00, v01, v10, v11)
    for k in range(KBITS):
        pred = ftv
        for j in range(4):
            hit = ((vs[j] >> k) & one) == one
            pred = jnp.minimum(pred, jnp.where(hit, tv[j], big))
        pred_v[pl.ds(16 * k, LANES)] = pred
        sym_v[pl.ds(16 * k, LANES)] = (st >> k) & one

    pltpu.sync_copy(pred_v, pred_hbm.at[pl.ds(base, ROWS)])
    pltpu.sync_copy(sym_v, sym_hbm.at[pl.ds(base, ROWS)])


def kernel(binary_list, eval, weight_initial, weights):
    pw = jnp.broadcast_to(
        jnp.concatenate([weight_initial.reshape(-1), weights.reshape(-1)])[:, None],
        (10, LANES),
    )
    pred, sym = _sc_walk(binary_list.T, pw)
    return pred, sym.reshape(B, 1, 1)
